# baseline (speedup 1.0000x reference)
import jax
import jax.numpy as jnp
from jax.experimental import pallas as pl
from jax.experimental.pallas import tpu as pltpu

SEQ_LEN = 2048
BT_SIZE = 4
D_MODEL = 4096


def _dma_kernel(x_ref, p_ref, o_ref, sem):
    copies = []
    for b in range(BT_SIZE):
        copies.append(
            pltpu.make_async_copy(
                x_ref.at[b], o_ref.at[b, :, pl.ds(0, D_MODEL)], sem.at[2 * b]
            )
        )
        copies.append(
            pltpu.make_async_copy(
                p_ref, o_ref.at[b, :, pl.ds(D_MODEL, 1)], sem.at[2 * b + 1]
            )
        )
    for c in copies:
        c.start()
    for c in copies:
        c.wait()


def kernel(inputs, pos_table):
    return pl.pallas_call(
        _dma_kernel,
        in_specs=[
            pl.BlockSpec(memory_space=pltpu.MemorySpace.HBM),
            pl.BlockSpec(memory_space=pltpu.MemorySpace.HBM),
        ],
        out_specs=pl.BlockSpec(memory_space=pltpu.MemorySpace.HBM),
        out_shape=jax.ShapeDtypeStruct((BT_SIZE, SEQ_LEN, D_MODEL + 1), jnp.float32),
        scratch_shapes=[pltpu.SemaphoreType.DMA((2 * BT_SIZE,))],
    )(inputs, pos_table)


# speedup vs baseline: 21.9344x; 21.9344x over previous
import jax
import jax.numpy as jnp
from jax.experimental import pallas as pl

SEQ_LEN = 2048
BT_SIZE = 4
D_MODEL = 4096
ROWS = SEQ_LEN * BT_SIZE


def _concat_kernel(x_ref, p_ref, o_ref):
    o_ref[:, :D_MODEL] = x_ref[...]
    o_ref[:, D_MODEL:] = p_ref[...]


def kernel(inputs, pos_table):
    R = 512
    x = inputs.reshape(ROWS, D_MODEL)
    out = pl.pallas_call(
        _concat_kernel,
        grid=(ROWS // R,),
        in_specs=[
            pl.BlockSpec((R, D_MODEL), lambda i: (i, 0)),
            pl.BlockSpec((R, 1), lambda i: (i % (SEQ_LEN // R), 0)),
        ],
        out_specs=pl.BlockSpec((R, D_MODEL + 1), lambda i: (i, 0)),
        out_shape=jax.ShapeDtypeStruct((ROWS, D_MODEL + 1), jnp.float32),
    )(x, pos_table)
    return out.reshape(BT_SIZE, SEQ_LEN, D_MODEL + 1)
